# Initial kernel scaffold; baseline (speedup 1.0000x reference)
import jax
import jax.numpy as jnp
from jax.experimental import pallas as pl


def kernel(x, segment_ids, token_table, pos_table, seg_table):
    raise NotImplementedError("write your pallas kernel here")


# speedup vs baseline: 5.4683x; 5.4683x over previous
import functools

import jax
import jax.numpy as jnp
from jax import lax
from jax.experimental import pallas as pl
from jax.experimental.pallas import tpu as pltpu
from jax.experimental.pallas import tpu_sc as plsc

_VOCAB = 100000
_MAXLEN = 200
_EMBED = 64
_BATCH = 4096
_N = _BATCH * _MAXLEN
_NC, _NS = 2, 16
_NW = _NC * _NS
_TPW = _N // _NW
_G = 128
_C = 512
_GPC = _C // _G
_NCH = _TPW // _C
_IDXROWS = _N // _G


def _sc_body(xf, psf, tok_hbm, ps_hbm, out_hbm, idx_x, idx_p, buf_a, buf_b,
             gsem):
    wid = lax.axis_index("s") * _NC + lax.axis_index("c")

    def chunk(c, carry):
        base = wid * _TPW + c * _C
        pltpu.sync_copy(xf.at[pl.ds(base, _C)], idx_x)
        pltpu.sync_copy(psf.at[pl.ds(base, _C)], idx_p)
        descs = []
        for g in range(_GPC):
            descs.append(pltpu.async_copy(
                tok_hbm.at[idx_x.at[pl.ds(g * _G, _G)]],
                buf_a.at[pl.ds(g * _G, _G)], gsem))
            descs.append(pltpu.async_copy(
                ps_hbm.at[idx_p.at[pl.ds(g * _G, _G)]],
                buf_b.at[pl.ds(g * _G, _G)], gsem))
        for d in descs:
            d.wait()

        def add_tok(t, carry2):
            for j in range(_EMBED // 16):
                col = pl.ds(j * 16, 16)
                plsc.addupdate(buf_a.at[t, col], buf_b[t, col])
            return carry2

        lax.fori_loop(0, _C, add_tok, 0, unroll=2)
        pltpu.sync_copy(buf_a, out_hbm.at[pl.ds(base, _C)])
        return carry

    lax.fori_loop(0, _NCH, chunk, 0)


@functools.partial(jax.jit, static_argnames=())
def _launch(xf, psf, token_table, ps_comb):
    mesh = plsc.VectorSubcoreMesh(core_axis_name="c", subcore_axis_name="s")
    return pl.kernel(
        _sc_body,
        out_type=jax.ShapeDtypeStruct((_N, _EMBED), jnp.float32),
        mesh=mesh,
        scratch_types=[
            pltpu.VMEM((_C,), jnp.int32),
            pltpu.VMEM((_C,), jnp.int32),
            pltpu.VMEM((_C, _EMBED), jnp.float32),
            pltpu.VMEM((_C, _EMBED), jnp.float32),
            pltpu.SemaphoreType.DMA,
        ],
        compiler_params=pltpu.CompilerParams(use_tc_tiling_on_sc=False),
    )(xf, psf, token_table, ps_comb)


def kernel(x, segment_ids, token_table, pos_table, seg_table):
    xf = x.astype(jnp.int32).reshape(_N)
    positions = jnp.arange(_MAXLEN, dtype=jnp.int32)
    psf = (segment_ids.astype(jnp.int32) * _MAXLEN
           + positions[None, :]).reshape(_N)
    ps_comb = (seg_table[:, None, :] + pos_table[None, :, :]).reshape(
        2 * _MAXLEN, _EMBED)
    out = _launch(xf, psf, token_table, ps_comb)
    return out.reshape(_BATCH, _MAXLEN, _EMBED)


# speedup vs baseline: 5.6032x; 1.0247x over previous
import functools

import jax
import jax.numpy as jnp
from jax import lax
from jax.experimental import pallas as pl
from jax.experimental.pallas import tpu as pltpu
from jax.experimental.pallas import tpu_sc as plsc

_VOCAB = 100000
_MAXLEN = 200
_EMBED = 64
_BATCH = 4096
_N = _BATCH * _MAXLEN
_NC, _NS = 2, 16
_NW = _NC * _NS
_TPW = _N // _NW
_G = 128
_C = 256
_GPC = _C // _G
_NCH = _TPW // _C


def _sc_body(xf, psf, tok_hbm, ps_hbm, out_hbm, idx_x, idx_p, buf_a, buf_b,
             isem0, isem1, gsem0, gsem1, osem0, osem1):
    wid = lax.axis_index("s") * _NC + lax.axis_index("c")
    isem = (isem0, isem1)
    gsem = (gsem0, gsem1)
    osem = (osem0, osem1)

    def do_idx(c, p):
        base = wid * _TPW + c * _C
        pltpu.async_copy(xf.at[pl.ds(base, _C)], idx_x.at[p], isem[p])
        pltpu.async_copy(psf.at[pl.ds(base, _C)], idx_p.at[p], isem[p])

    def wait_idx(p):
        pltpu.make_async_copy(xf.at[pl.ds(0, _C)], idx_x.at[p],
                              isem[p]).wait()
        pltpu.make_async_copy(psf.at[pl.ds(0, _C)], idx_p.at[p],
                              isem[p]).wait()

    def do_gather(p):
        for g in range(_GPC):
            rows = pl.ds(g * _G, _G)
            pltpu.async_copy(tok_hbm.at[idx_x.at[p, rows]],
                             buf_a.at[p, rows], gsem[p])
            pltpu.async_copy(ps_hbm.at[idx_p.at[p, rows]],
                             buf_b.at[p, rows], gsem[p])

    def wait_gather(p):
        for g in range(_GPC):
            rows = pl.ds(g * _G, _G)
            pltpu.make_async_copy(tok_hbm.at[pl.ds(0, _G)],
                                  buf_a.at[p, rows], gsem[p]).wait()
            pltpu.make_async_copy(tok_hbm.at[pl.ds(0, _G)],
                                  buf_b.at[p, rows], gsem[p]).wait()

    def do_scatter(c, p):
        base = wid * _TPW + c * _C
        pltpu.async_copy(buf_a.at[p], out_hbm.at[pl.ds(base, _C)], osem[p])

    def wait_scatter(p):
        pltpu.make_async_copy(buf_a.at[p], out_hbm.at[pl.ds(0, _C)],
                              osem[p]).wait()

    do_idx(0, 0)
    wait_idx(0)
    do_gather(0)
    do_idx(1, 1)

    def half(c, p):
        q = 1 - p

        @pl.when(c + 1 < _NCH)
        def _():
            wait_idx(q)

            @pl.when(c >= 1)
            def _():
                wait_scatter(q)

            do_gather(q)

        wait_gather(p)

        @pl.when(c + 2 < _NCH)
        def _():
            do_idx(c + 2, p)

        def add_tok(t, carry2):
            for j in range(_EMBED // 16):
                col = pl.ds(j * 16, 16)
                plsc.addupdate(buf_a.at[p, t, col], buf_b[p, t, col])
            return carry2

        lax.fori_loop(0, _C, add_tok, 0, unroll=4)
        do_scatter(c, p)

    def pair(c2, carry):
        half(2 * c2, 0)
        half(2 * c2 + 1, 1)
        return carry

    lax.fori_loop(0, _NCH // 2, pair, 0)
    wait_scatter(0)
    wait_scatter(1)


@functools.partial(jax.jit, static_argnames=())
def _launch(xf, psf, token_table, ps_comb):
    mesh = plsc.VectorSubcoreMesh(core_axis_name="c", subcore_axis_name="s")
    return pl.kernel(
        _sc_body,
        out_type=jax.ShapeDtypeStruct((_N, _EMBED), jnp.float32),
        mesh=mesh,
        scratch_types=[
            pltpu.VMEM((2, _C), jnp.int32),
            pltpu.VMEM((2, _C), jnp.int32),
            pltpu.VMEM((2, _C, _EMBED), jnp.float32),
            pltpu.VMEM((2, _C, _EMBED), jnp.float32),
            pltpu.SemaphoreType.DMA,
            pltpu.SemaphoreType.DMA,
            pltpu.SemaphoreType.DMA,
            pltpu.SemaphoreType.DMA,
            pltpu.SemaphoreType.DMA,
            pltpu.SemaphoreType.DMA,
        ],
        compiler_params=pltpu.CompilerParams(use_tc_tiling_on_sc=False),
    )(xf, psf, token_table, ps_comb)


def kernel(x, segment_ids, token_table, pos_table, seg_table):
    xf = x.astype(jnp.int32).reshape(_N)
    positions = jnp.arange(_MAXLEN, dtype=jnp.int32)
    psf = (segment_ids.astype(jnp.int32) * _MAXLEN
           + positions[None, :]).reshape(_N)
    ps_comb = (seg_table[:, None, :] + pos_table[None, :, :]).reshape(
        2 * _MAXLEN, _EMBED)
    out = _launch(xf, psf, token_table, ps_comb)
    return out.reshape(_BATCH, _MAXLEN, _EMBED)
